# baseline (speedup 1.0000x reference)
import functools

import jax
import jax.numpy as jnp
from jax import lax
from jax.experimental import pallas as pl
from jax.experimental.pallas import tpu as pltpu
from jax.experimental.pallas import tpu_sc as plsc

N = 50000
E = 800000
ET = E + N
NBLK = 6656
NBLK_S = NBLK + 8
NR = N + 48
RPT = NR // 16
BPT1 = NBLK // 16
BPT2 = NBLK // 32
K = 8
SUP1 = BPT1 // K
SUP2 = BPT2 // K
AHEAD = 3
NBUF = 4

_mesh = plsc.VectorSubcoreMesh(core_axis_name="c", subcore_axis_name="s")
_sc_params = pltpu.CompilerParams(use_tc_tiling_on_sc=False)


def _superstep(sup, parity, wbase, srce, dste, table, acc, sidx, didx, rows,
               gsem):
    rowbase = wbase + sup * K
    pltpu.sync_copy(srce.at[pl.ds(rowbase + K, K)], sidx[1 - parity])
    pltpu.sync_copy(dste.at[pl.ds(rowbase + K, K)], didx[1 - parity])
    for j in range(K):
        jj = j + AHEAD
        sb, rw = (sidx[parity], jj) if jj < K else (sidx[1 - parity], jj - K)
        b = jj % NBUF
        pltpu.async_copy(table.at[sb.at[rw]], rows[b], gsem[b])
        b = j % NBUF
        pltpu.make_async_copy(table.at[sidx[parity].at[j]], rows[b],
                              gsem[b]).wait()
        pltpu.sync_copy(rows[b], acc.at[didx[parity].at[j]], add=True)


def _prologue(wbase, srce, dste, table, sidx, didx, rows, gsem):
    pltpu.sync_copy(srce.at[pl.ds(wbase, K)], sidx[0])
    pltpu.sync_copy(dste.at[pl.ds(wbase, K)], didx[0])
    for r in range(AHEAD):
        pltpu.async_copy(table.at[sidx[0].at[r]], rows[r], gsem[r])


def _drain(table, sidx, rows, gsem, nblocks):
    for r in range(AHEAD):
        b = (nblocks + r) % NBUF
        pltpu.make_async_copy(table.at[sidx[0].at[r]], rows[b], gsem[b]).wait()


@functools.partial(
    pl.kernel,
    out_type=jax.ShapeDtypeStruct((2, NR, 16), jnp.float32),
    mesh=_mesh,
    compiler_params=_sc_params,
    scratch_types=[
        pltpu.VMEM_SHARED((NR, 16), jnp.float32),
        pltpu.VMEM((K, 128), jnp.int32),
        pltpu.VMEM((128, 16), jnp.float32),
    ],
)
def _sc_degree(dste, zeros16, ones16, out, acc, didx, ones_v):
    c = lax.axis_index("c")
    s = lax.axis_index("s")
    pltpu.sync_copy(ones16, ones_v)
    pltpu.sync_copy(zeros16, acc.at[pl.ds(s * RPT, RPT)])
    plsc.subcore_barrier()

    def body(sup, carry):
        rowbase = (c * 16 + s) * BPT2 + sup * K
        pltpu.sync_copy(dste.at[pl.ds(rowbase, K)], didx)
        for j in range(K):
            pltpu.sync_copy(ones_v, acc.at[didx.at[j]], add=True)
        return carry

    lax.fori_loop(0, SUP2, body, 0)
    plsc.subcore_barrier()
    pltpu.sync_copy(acc.at[pl.ds(s * RPT, RPT)], out.at[c, pl.ds(s * RPT, RPT)])


@functools.partial(
    pl.kernel,
    out_type=[jax.ShapeDtypeStruct((NR, 32), jnp.float32) for _ in range(4)],
    mesh=_mesh,
    compiler_params=_sc_params,
    scratch_types=[
        pltpu.VMEM_SHARED((NR, 32), jnp.float32),
        pltpu.VMEM((K, 128), jnp.int32),
        pltpu.VMEM((K, 128), jnp.int32),
        pltpu.VMEM((K, 128), jnp.int32),
        pltpu.VMEM((K, 128), jnp.int32),
    ]
    + [pltpu.VMEM((128, 32), jnp.float32) for _ in range(NBUF)]
    + [pltpu.SemaphoreType.DMA for _ in range(NBUF)],
)
def _sc_layer1(srce, dste, xs0, xs1, xs2, xs3, zeros32,
               o0, o1, o2, o3, acc, si0, si1, di0, di1,
               r0, r1, r2, r3, g0, g1, g2, g3):
    c = lax.axis_index("c")
    s = lax.axis_index("s")
    sidx, didx = (si0, si1), (di0, di1)
    rows, gsem = (r0, r1, r2, r3), (g0, g1, g2, g3)
    tables = (xs0, xs1, xs2, xs3)
    outs = (o0, o1, o2, o3)
    for g in range(4):
        @pl.when(c == g // 2)
        def _(g=g):
            table, out = tables[g], outs[g]
            pltpu.sync_copy(zeros32, acc.at[pl.ds(s * RPT, RPT)])
            plsc.subcore_barrier()
            wbase = s * BPT1
            _prologue(wbase, srce, dste, table, sidx, didx, rows, gsem)

            def body(sup2, carry):
                _superstep(2 * sup2, 0, wbase, srce, dste, table, acc,
                           sidx, didx, rows, gsem)
                _superstep(2 * sup2 + 1, 1, wbase, srce, dste, table, acc,
                           sidx, didx, rows, gsem)
                return carry

            lax.fori_loop(0, SUP1 // 2, body, 0)
            _drain(table, sidx, rows, gsem, BPT1)
            plsc.subcore_barrier()
            pltpu.sync_copy(acc.at[pl.ds(s * RPT, RPT)],
                            out.at[pl.ds(s * RPT, RPT)])
            plsc.subcore_barrier()


@functools.partial(
    pl.kernel,
    out_type=jax.ShapeDtypeStruct((2, NR, 16), jnp.float32),
    mesh=_mesh,
    compiler_params=_sc_params,
    scratch_types=[
        pltpu.VMEM_SHARED((NR, 16), jnp.float32),
        pltpu.VMEM((K, 128), jnp.int32),
        pltpu.VMEM((K, 128), jnp.int32),
        pltpu.VMEM((K, 128), jnp.int32),
        pltpu.VMEM((K, 128), jnp.int32),
    ]
    + [pltpu.VMEM((128, 16), jnp.float32) for _ in range(NBUF)]
    + [pltpu.SemaphoreType.DMA for _ in range(NBUF)],
)
def _sc_layer2(srce, dste, y16, zeros16, out, acc, si0, si1, di0, di1,
               r0, r1, r2, r3, g0, g1, g2, g3):
    c = lax.axis_index("c")
    s = lax.axis_index("s")
    sidx, didx = (si0, si1), (di0, di1)
    rows, gsem = (r0, r1, r2, r3), (g0, g1, g2, g3)
    pltpu.sync_copy(zeros16, acc.at[pl.ds(s * RPT, RPT)])
    plsc.subcore_barrier()
    wbase = (c * 16 + s) * BPT2
    _prologue(wbase, srce, dste, y16, sidx, didx, rows, gsem)

    def body(sup2, carry):
        _superstep(2 * sup2, 0, wbase, srce, dste, y16, acc,
                   sidx, didx, rows, gsem)
        _superstep(2 * sup2 + 1, 1, wbase, srce, dste, y16, acc,
                   sidx, didx, rows, gsem)
        return carry

    lax.fori_loop(0, SUP2 // 2, body, 0)
    _drain(y16, sidx, rows, gsem, BPT2)
    plsc.subcore_barrier()
    pltpu.sync_copy(acc.at[pl.ds(s * RPT, RPT)], out.at[c, pl.ds(s * RPT, RPT)])


BN = 2000


def _tc_scale_body(inp_ref, degp_ref, xs0, xs1, xs2, xs3, scl_ref):
    deg = degp_ref[0, :, 0] + degp_ref[1, :, 0]
    isq = lax.rsqrt(deg)
    for l, xs in enumerate((xs0, xs1, xs2, xs3)):
        xl = inp_ref[l, :, 125:150]
        xs[...] = jnp.pad(xl * isq[:, None], ((0, 0), (0, 7)))
    scl_ref[...] = jnp.stack([isq, isq], axis=1)


def _tc_dense_body(a0, a1, a2, a3, scl_ref, w1_ref, w2_ref, y16_ref):
    isq = scl_ref[:, 0]
    acc = jnp.dot(a0[...] * isq[:, None], w1_ref[0],
                  preferred_element_type=jnp.float32)
    for l, a in enumerate((a1, a2, a3)):
        acc = acc + jnp.dot(a[...] * isq[:, None], w1_ref[l + 1],
                            preferred_element_type=jnp.float32)
    h = jnp.maximum(acc, 0.0)
    y = jnp.dot(h, w2_ref[...], preferred_element_type=jnp.float32)
    y16_ref[...] = jnp.pad(y * isq[:, None], ((0, 0), (0, 14)))


def _tc_final_body(aggp_ref, scl_ref, out_ref):
    agg2 = aggp_ref[0, :, 0:2] + aggp_ref[1, :, 0:2]
    out_ref[...] = agg2 * scl_ref[:, 0][:, None]


def kernel(input_list, ts_list, edge_index, W1, W2):
    xr = input_list.reshape(4, N, 200)
    npad = NBLK_S * 128 - ET
    selfe = lax.iota(jnp.int32, N)
    src = jnp.concatenate(
        [edge_index[0], selfe,
         jnp.zeros((npad,), jnp.int32)]).reshape(NBLK_S, 128)
    dst = jnp.concatenate(
        [edge_index[1], selfe,
         jnp.full((npad,), N, jnp.int32)]).reshape(NBLK_S, 128)
    zeros32 = jnp.zeros((RPT, 32), jnp.float32)
    zeros16 = jnp.zeros((RPT, 16), jnp.float32)
    ones16 = jnp.ones((128, 16), jnp.float32)
    w1r = jnp.pad(W1.reshape(4, 25, 100), ((0, 0), (0, 7), (0, 0)))

    degp = _sc_degree(dst, zeros16, ones16)

    xs0, xs1, xs2, xs3, scl = pl.pallas_call(
        _tc_scale_body,
        grid=(N // BN,),
        in_specs=[
            pl.BlockSpec((4, BN, 200), lambda i: (0, i, 0)),
            pl.BlockSpec((2, BN, 16), lambda i: (0, i, 0)),
        ],
        out_specs=[pl.BlockSpec((BN, 32), lambda i: (i, 0)) for _ in range(4)]
        + [pl.BlockSpec((BN, 2), lambda i: (i, 0))],
        out_shape=[jax.ShapeDtypeStruct((N, 32), jnp.float32) for _ in range(4)]
        + [jax.ShapeDtypeStruct((N, 2), jnp.float32)],
    )(xr, degp)

    agg = _sc_layer1(src, dst, xs0, xs1, xs2, xs3, zeros32)

    y16 = pl.pallas_call(
        _tc_dense_body,
        grid=(N // BN,),
        in_specs=[pl.BlockSpec((BN, 32), lambda i: (i, 0)) for _ in range(4)]
        + [
            pl.BlockSpec((BN, 2), lambda i: (i, 0)),
            pl.BlockSpec((4, 32, 100), lambda i: (0, 0, 0)),
            pl.BlockSpec((100, 2), lambda i: (0, 0)),
        ],
        out_specs=pl.BlockSpec((BN, 16), lambda i: (i, 0)),
        out_shape=jax.ShapeDtypeStruct((N, 16), jnp.float32),
    )(agg[0], agg[1], agg[2], agg[3], scl, w1r, W2)

    agg2p = _sc_layer2(src, dst, y16, zeros16)

    out = pl.pallas_call(
        _tc_final_body,
        grid=(N // BN,),
        in_specs=[
            pl.BlockSpec((2, BN, 16), lambda i: (0, i, 0)),
            pl.BlockSpec((BN, 2), lambda i: (i, 0)),
        ],
        out_specs=pl.BlockSpec((BN, 2), lambda i: (i, 0)),
        out_shape=jax.ShapeDtypeStruct((N, 2), jnp.float32),
    )(agg2p, scl)
    return out


# speedup vs baseline: 1.2382x; 1.2382x over previous
import functools

import jax
import jax.numpy as jnp
from jax import lax
from jax.experimental import pallas as pl
from jax.experimental.pallas import tpu as pltpu
from jax.experimental.pallas import tpu_sc as plsc

N = 50000
E = 800000
ET = E + N
NBLK = 6656
NBLK_S = NBLK + 8
NR = N + 48
RPT = NR // 16
BPT1 = NBLK // 16
BPT2 = NBLK // 32
K = 8
SUP1 = BPT1 // K
SUP2 = BPT2 // K
AHEAD = 3
NBUF = 4

_mesh = plsc.VectorSubcoreMesh(core_axis_name="c", subcore_axis_name="s")
_sc_params = pltpu.CompilerParams(use_tc_tiling_on_sc=False)


def _superstep(sup, parity, wbase, srce, dste, table, acc, sidx, didx, rows,
               gsem):
    rowbase = wbase + sup * K
    pltpu.sync_copy(srce.at[pl.ds(rowbase + K, K)], sidx[1 - parity])
    pltpu.sync_copy(dste.at[pl.ds(rowbase + K, K)], didx[1 - parity])
    for j in range(K):
        jj = j + AHEAD
        sb, rw = (sidx[parity], jj) if jj < K else (sidx[1 - parity], jj - K)
        b = jj % NBUF
        pltpu.async_copy(table.at[sb.at[rw]], rows[b], gsem[b])
        b = j % NBUF
        pltpu.make_async_copy(table.at[sidx[parity].at[j]], rows[b],
                              gsem[b]).wait()
        pltpu.sync_copy(rows[b], acc.at[didx[parity].at[j]], add=True)


def _prologue(wbase, srce, dste, table, sidx, didx, rows, gsem):
    pltpu.sync_copy(srce.at[pl.ds(wbase, K)], sidx[0])
    pltpu.sync_copy(dste.at[pl.ds(wbase, K)], didx[0])
    for r in range(AHEAD):
        pltpu.async_copy(table.at[sidx[0].at[r]], rows[r], gsem[r])


def _drain(table, sidx, rows, gsem, nblocks):
    for r in range(AHEAD):
        b = (nblocks + r) % NBUF
        pltpu.make_async_copy(table.at[sidx[0].at[r]], rows[b], gsem[b]).wait()


@functools.partial(
    pl.kernel,
    out_type=jax.ShapeDtypeStruct((2, NR, 16), jnp.float32),
    mesh=_mesh,
    compiler_params=_sc_params,
    scratch_types=[
        pltpu.VMEM_SHARED((NR, 16), jnp.float32),
        pltpu.VMEM((K, 128), jnp.int32),
        pltpu.VMEM((128, 16), jnp.float32),
    ],
)
def _sc_degree(dste, zeros16, ones16, out, acc, didx, ones_v):
    c = lax.axis_index("c")
    s = lax.axis_index("s")
    pltpu.sync_copy(ones16, ones_v)
    pltpu.sync_copy(zeros16, acc.at[pl.ds(s * RPT, RPT)])
    plsc.subcore_barrier()

    def body(sup, carry):
        rowbase = (c * 16 + s) * BPT2 + sup * K
        pltpu.sync_copy(dste.at[pl.ds(rowbase, K)], didx)
        for j in range(K):
            pltpu.sync_copy(ones_v, acc.at[didx.at[j]], add=True)
        return carry

    lax.fori_loop(0, SUP2, body, 0)
    plsc.subcore_barrier()
    pltpu.sync_copy(acc.at[pl.ds(s * RPT, RPT)], out.at[c, pl.ds(s * RPT, RPT)])


@functools.partial(
    pl.kernel,
    out_type=[jax.ShapeDtypeStruct((NR, 32), jnp.float32) for _ in range(4)],
    mesh=_mesh,
    compiler_params=_sc_params,
    scratch_types=[
        pltpu.VMEM_SHARED((NR, 32), jnp.float32),
        pltpu.VMEM((K, 128), jnp.int32),
        pltpu.VMEM((K, 128), jnp.int32),
        pltpu.VMEM((K, 128), jnp.int32),
        pltpu.VMEM((K, 128), jnp.int32),
    ]
    + [pltpu.VMEM((128, 32), jnp.float32) for _ in range(NBUF)]
    + [pltpu.SemaphoreType.DMA for _ in range(NBUF)],
)
def _sc_layer1(srce, dste, xs0, xs1, xs2, xs3, zeros32,
               o0, o1, o2, o3, acc, si0, si1, di0, di1,
               r0, r1, r2, r3, g0, g1, g2, g3):
    c = lax.axis_index("c")
    s = lax.axis_index("s")
    sidx, didx = (si0, si1), (di0, di1)
    rows, gsem = (r0, r1, r2, r3), (g0, g1, g2, g3)
    tables = (xs0, xs1, xs2, xs3)
    outs = (o0, o1, o2, o3)
    for g in range(4):
        @pl.when(c == g // 2)
        def _(g=g):
            table, out = tables[g], outs[g]
            pltpu.sync_copy(zeros32, acc.at[pl.ds(s * RPT, RPT)])
            plsc.subcore_barrier()
            wbase = s * BPT1
            _prologue(wbase, srce, dste, table, sidx, didx, rows, gsem)

            def body(sup2, carry):
                _superstep(2 * sup2, 0, wbase, srce, dste, table, acc,
                           sidx, didx, rows, gsem)
                _superstep(2 * sup2 + 1, 1, wbase, srce, dste, table, acc,
                           sidx, didx, rows, gsem)
                return carry

            lax.fori_loop(0, SUP1 // 2, body, 0)
            _drain(table, sidx, rows, gsem, BPT1)
            plsc.subcore_barrier()
            pltpu.sync_copy(acc.at[pl.ds(s * RPT, RPT)],
                            out.at[pl.ds(s * RPT, RPT)])
            plsc.subcore_barrier()


@functools.partial(
    pl.kernel,
    out_type=jax.ShapeDtypeStruct((2, NR, 16), jnp.float32),
    mesh=_mesh,
    compiler_params=_sc_params,
    scratch_types=[
        pltpu.VMEM_SHARED((NR, 16), jnp.float32),
        pltpu.VMEM((K, 128), jnp.int32),
        pltpu.VMEM((K, 128), jnp.int32),
        pltpu.VMEM((K, 128), jnp.int32),
        pltpu.VMEM((K, 128), jnp.int32),
    ]
    + [pltpu.VMEM((128, 16), jnp.float32) for _ in range(NBUF)]
    + [pltpu.SemaphoreType.DMA for _ in range(NBUF)],
)
def _sc_layer2(srce, dste, y16, zeros16, out, acc, si0, si1, di0, di1,
               r0, r1, r2, r3, g0, g1, g2, g3):
    c = lax.axis_index("c")
    s = lax.axis_index("s")
    sidx, didx = (si0, si1), (di0, di1)
    rows, gsem = (r0, r1, r2, r3), (g0, g1, g2, g3)
    pltpu.sync_copy(zeros16, acc.at[pl.ds(s * RPT, RPT)])
    plsc.subcore_barrier()
    wbase = (c * 16 + s) * BPT2
    _prologue(wbase, srce, dste, y16, sidx, didx, rows, gsem)

    def body(sup2, carry):
        _superstep(2 * sup2, 0, wbase, srce, dste, y16, acc,
                   sidx, didx, rows, gsem)
        _superstep(2 * sup2 + 1, 1, wbase, srce, dste, y16, acc,
                   sidx, didx, rows, gsem)
        return carry

    lax.fori_loop(0, SUP2 // 2, body, 0)
    _drain(y16, sidx, rows, gsem, BPT2)
    plsc.subcore_barrier()
    pltpu.sync_copy(acc.at[pl.ds(s * RPT, RPT)], out.at[c, pl.ds(s * RPT, RPT)])


BN = 2000


def _tc_scale_body(inp_ref, degp_ref, xs0, xs1, xs2, xs3, scl_ref):
    deg = degp_ref[0, :, 0] + degp_ref[1, :, 0]
    isq = lax.rsqrt(deg)
    for l, xs in enumerate((xs0, xs1, xs2, xs3)):
        xs[...] = jnp.pad(inp_ref[l, :, :] * isq[:, None], ((0, 0), (0, 7)))
    scl_ref[...] = jnp.stack([isq, isq], axis=1)


def _tc_dense_body(a0, a1, a2, a3, scl_ref, w1_ref, w2_ref, y16_ref):
    isq = scl_ref[:, 0]
    acc = jnp.dot(a0[...] * isq[:, None], w1_ref[0],
                  preferred_element_type=jnp.float32)
    for l, a in enumerate((a1, a2, a3)):
        acc = acc + jnp.dot(a[...] * isq[:, None], w1_ref[l + 1],
                            preferred_element_type=jnp.float32)
    h = jnp.maximum(acc, 0.0)
    y = jnp.dot(h, w2_ref[...], preferred_element_type=jnp.float32)
    y16_ref[...] = jnp.pad(y * isq[:, None], ((0, 0), (0, 14)))


def _tc_final_body(aggp_ref, scl_ref, out_ref):
    agg2 = aggp_ref[0, :, 0:2] + aggp_ref[1, :, 0:2]
    out_ref[...] = agg2 * scl_ref[:, 0][:, None]


def kernel(input_list, ts_list, edge_index, W1, W2):
    xsl = input_list[:, :, -3, :]
    npad = NBLK_S * 128 - ET
    selfe = lax.iota(jnp.int32, N)
    src = jnp.concatenate(
        [edge_index[0], selfe,
         jnp.zeros((npad,), jnp.int32)]).reshape(NBLK_S, 128)
    dst = jnp.concatenate(
        [edge_index[1], selfe,
         jnp.full((npad,), N, jnp.int32)]).reshape(NBLK_S, 128)
    zeros32 = jnp.zeros((RPT, 32), jnp.float32)
    zeros16 = jnp.zeros((RPT, 16), jnp.float32)
    ones16 = jnp.ones((128, 16), jnp.float32)
    w1r = jnp.pad(W1.reshape(4, 25, 100), ((0, 0), (0, 7), (0, 0)))

    degp = _sc_degree(dst, zeros16, ones16)

    xs0, xs1, xs2, xs3, scl = pl.pallas_call(
        _tc_scale_body,
        grid=(N // BN,),
        in_specs=[
            pl.BlockSpec((4, BN, 25), lambda i: (0, i, 0)),
            pl.BlockSpec((2, BN, 16), lambda i: (0, i, 0)),
        ],
        out_specs=[pl.BlockSpec((BN, 32), lambda i: (i, 0)) for _ in range(4)]
        + [pl.BlockSpec((BN, 2), lambda i: (i, 0))],
        out_shape=[jax.ShapeDtypeStruct((N, 32), jnp.float32) for _ in range(4)]
        + [jax.ShapeDtypeStruct((N, 2), jnp.float32)],
    )(xsl, degp)

    agg = _sc_layer1(src, dst, xs0, xs1, xs2, xs3, zeros32)

    y16 = pl.pallas_call(
        _tc_dense_body,
        grid=(N // BN,),
        in_specs=[pl.BlockSpec((BN, 32), lambda i: (i, 0)) for _ in range(4)]
        + [
            pl.BlockSpec((BN, 2), lambda i: (i, 0)),
            pl.BlockSpec((4, 32, 100), lambda i: (0, 0, 0)),
            pl.BlockSpec((100, 2), lambda i: (0, 0)),
        ],
        out_specs=pl.BlockSpec((BN, 16), lambda i: (i, 0)),
        out_shape=jax.ShapeDtypeStruct((N, 16), jnp.float32),
    )(agg[0], agg[1], agg[2], agg[3], scl, w1r, W2)

    agg2p = _sc_layer2(src, dst, y16, zeros16)

    out = pl.pallas_call(
        _tc_final_body,
        grid=(N // BN,),
        in_specs=[
            pl.BlockSpec((2, BN, 16), lambda i: (0, i, 0)),
            pl.BlockSpec((BN, 2), lambda i: (i, 0)),
        ],
        out_specs=pl.BlockSpec((BN, 2), lambda i: (i, 0)),
        out_shape=jax.ShapeDtypeStruct((N, 2), jnp.float32),
    )(agg2p, scl)
    return out
